# baseline (speedup 1.0000x reference)
import jax
import jax.numpy as jnp
from jax import lax
from jax.experimental import pallas as pl
from jax.experimental.pallas import tpu as pltpu
from jax.experimental.pallas import tpu_sc as plsc

_NUM_ROWS = 128
_NUM_COLS = 128
_NC = 1
_NS = 16
_NW = _NC * _NS
_RPW = _NUM_ROWS // _NW
_HALF = _RPW // 2
_L = 16
_CH = _NUM_COLS // _L


def _sc_body(nb_hbm, idx_hbm, out_hbm, nb_v, idx_v, out_v, sem_nb, sem_a):
    wid = lax.axis_index("s") * _NC + lax.axis_index("c")
    base = wid * _RPW
    c_nb = pltpu.async_copy(nb_hbm, nb_v.at[pl.ds(0, _NUM_ROWS)], sem_nb)
    c_a = pltpu.async_copy(idx_hbm.at[pl.ds(base, _RPW)], idx_v, sem_a)

    zeros = jnp.zeros((_L,), jnp.int32)
    ones = jnp.ones((_L,), jnp.int32)
    lanes = lax.broadcasted_iota(jnp.int32, (_L,), 0)

    def zero_chunk(i, carry):
        out_v[i // _CH, pl.ds((i % _CH) * _L, _L)] = zeros
        return carry

    lax.fori_loop(0, _RPW * _CH, zero_chunk, 0, unroll=4)

    def scatter_row(rl, carry):
        nb_r = nb_v[pl.ds(base + rl, _L)][0]
        rows = jnp.full((_L,), rl, jnp.int32)

        def chunk(g, c2):
            mask = (lanes + g * _L) < nb_r
            ids = idx_v[rl, pl.ds(g * _L, _L)]
            plsc.store_scatter(out_v, [rows, ids], ones, mask=mask)
            return c2

        lax.fori_loop(0, _CH, chunk, 0, unroll=2)
        return carry

    c_nb.wait()
    c_a.wait()
    lax.fori_loop(0, _RPW, scatter_row, 0)
    pltpu.sync_copy(out_v, out_hbm.at[pl.ds(base, _RPW)])


def kernel(kv_num_blocks, kv_indices):
    mesh = plsc.VectorSubcoreMesh(
        core_axis_name="c", subcore_axis_name="s", num_cores=_NC)
    f = pl.kernel(
        _sc_body,
        out_type=jax.ShapeDtypeStruct((_NUM_ROWS, _NUM_COLS), jnp.int32),
        mesh=mesh,
        scratch_types=[
            pltpu.VMEM((_NUM_ROWS + _L,), jnp.int32),
            pltpu.VMEM((_RPW, _NUM_COLS), jnp.int32),
            pltpu.VMEM((_RPW, _NUM_COLS), jnp.int32),
            pltpu.SemaphoreType.DMA,
            pltpu.SemaphoreType.DMA,
        ],
        compiler_params=pltpu.CompilerParams(
            needs_layout_passes=False,
            disable_bounds_checks=True,
            disable_semaphore_checks=True,
        ),
    )
    return f(kv_num_blocks, kv_indices)


# speedup vs baseline: 1.0114x; 1.0114x over previous
import jax
import jax.numpy as jnp
from jax import lax
from jax.experimental import pallas as pl
from jax.experimental.pallas import tpu as pltpu
from jax.experimental.pallas import tpu_sc as plsc

_NUM_ROWS = 128
_NUM_COLS = 128
_NC = 1
_NS = 16
_NW = _NC * _NS
_RPW = _NUM_ROWS // _NW
_HALF = _RPW // 2
_L = 16
_CH = _NUM_COLS // _L


def _sc_body(nb_hbm, idx_hbm, out_hbm, nb_v, idx_v, out_v, sem_nb, sem_a):
    wid = lax.axis_index("s") * _NC + lax.axis_index("c")
    base = wid * _RPW
    c_nb = pltpu.async_copy(nb_hbm, nb_v.at[pl.ds(0, _NUM_ROWS)], sem_nb)
    c_a = pltpu.async_copy(idx_hbm.at[pl.ds(base, _RPW)], idx_v, sem_a)

    zeros = jnp.zeros((_L,), jnp.int32)
    ones = jnp.ones((_L,), jnp.int32)
    lanes = lax.broadcasted_iota(jnp.int32, (_L,), 0)

    @plsc.parallel_loop(0, _RPW * _CH, unroll=8)
    def _zero(i):
        out_v[i // _CH, pl.ds((i % _CH) * _L, _L)] = zeros

    c_nb.wait()
    c_a.wait()

    def scatter_row(rl, carry):
        nb_r = nb_v[pl.ds(base + rl, _L)][0]
        rows = jnp.full((_L,), rl, jnp.int32)

        @plsc.parallel_loop(0, _CH, unroll=4)
        def _chunk(g):
            mask = (lanes + g * _L) < nb_r
            ids = idx_v[rl, pl.ds(g * _L, _L)]
            plsc.store_scatter(out_v, [rows, ids], ones, mask=mask)

        return carry

    lax.fori_loop(0, _RPW, scatter_row, 0)
    pltpu.sync_copy(out_v, out_hbm.at[pl.ds(base, _RPW)])


def kernel(kv_num_blocks, kv_indices):
    mesh = plsc.VectorSubcoreMesh(
        core_axis_name="c", subcore_axis_name="s", num_cores=_NC)
    f = pl.kernel(
        _sc_body,
        out_type=jax.ShapeDtypeStruct((_NUM_ROWS, _NUM_COLS), jnp.int32),
        mesh=mesh,
        scratch_types=[
            pltpu.VMEM((_NUM_ROWS + _L,), jnp.int32),
            pltpu.VMEM((_RPW, _NUM_COLS), jnp.int32),
            pltpu.VMEM((_RPW, _NUM_COLS), jnp.int32),
            pltpu.SemaphoreType.DMA,
            pltpu.SemaphoreType.DMA,
        ],
        compiler_params=pltpu.CompilerParams(needs_layout_passes=False),
    )
    return f(kv_num_blocks, kv_indices)


# speedup vs baseline: 1.0161x; 1.0046x over previous
import jax
import jax.numpy as jnp
from jax import lax
from jax.experimental import pallas as pl
from jax.experimental.pallas import tpu as pltpu
from jax.experimental.pallas import tpu_sc as plsc

_NUM_ROWS = 128
_NUM_COLS = 128
_NC = 1
_NS = 16
_NW = _NC * _NS
_RPW = _NUM_ROWS // _NW
_HALF = _RPW // 2
_L = 16
_CH = _NUM_COLS // _L


def _sc_body(nb_hbm, idx_hbm, out_hbm, nb_v, idx_v, out_v, sem_nb, sem_a):
    wid = lax.axis_index("s") * _NC + lax.axis_index("c")
    base = wid * _RPW
    c_nb = pltpu.async_copy(nb_hbm, nb_v.at[pl.ds(0, _NUM_ROWS)], sem_nb)
    c_a = pltpu.async_copy(idx_hbm.at[pl.ds(base, _RPW)], idx_v, sem_a)

    zeros = jnp.zeros((_L,), jnp.int32)
    ones = jnp.ones((_L,), jnp.int32)
    lanes = lax.broadcasted_iota(jnp.int32, (_L,), 0)

    @plsc.parallel_loop(0, _RPW * _CH, unroll=8)
    def _zero(i):
        out_v[i // _CH, pl.ds((i % _CH) * _L, _L)] = zeros

    c_nb.wait()
    c_a.wait()

    @plsc.parallel_loop(0, _RPW * _CH, unroll=8)
    def _scatter(i):
        rl = i // _CH
        g = i % _CH
        nb_r = nb_v[pl.ds(base + rl, _L)][0]
        rows = jnp.full((_L,), 1, jnp.int32) * rl
        mask = (lanes + g * _L) < nb_r
        ids = idx_v[rl, pl.ds(g * _L, _L)]
        plsc.store_scatter(out_v, [rows, ids], ones, mask=mask)
    pltpu.sync_copy(out_v, out_hbm.at[pl.ds(base, _RPW)])


def kernel(kv_num_blocks, kv_indices):
    mesh = plsc.VectorSubcoreMesh(
        core_axis_name="c", subcore_axis_name="s", num_cores=_NC)
    f = pl.kernel(
        _sc_body,
        out_type=jax.ShapeDtypeStruct((_NUM_ROWS, _NUM_COLS), jnp.int32),
        mesh=mesh,
        scratch_types=[
            pltpu.VMEM((_NUM_ROWS + _L,), jnp.int32),
            pltpu.VMEM((_RPW, _NUM_COLS), jnp.int32),
            pltpu.VMEM((_RPW, _NUM_COLS), jnp.int32),
            pltpu.SemaphoreType.DMA,
            pltpu.SemaphoreType.DMA,
        ],
        compiler_params=pltpu.CompilerParams(needs_layout_passes=False),
    )
    return f(kv_num_blocks, kv_indices)


# speedup vs baseline: 1.0182x; 1.0021x over previous
import jax
import jax.numpy as jnp
from jax import lax
from jax.experimental import pallas as pl
from jax.experimental.pallas import tpu as pltpu
from jax.experimental.pallas import tpu_sc as plsc

_NUM_ROWS = 128
_NUM_COLS = 128
_NC = 1
_NS = 16
_NW = _NC * _NS
_RPW = _NUM_ROWS // _NW
_HALF = _RPW // 2
_L = 16
_CH = _NUM_COLS // _L


def _sc_body(nb_hbm, idx_hbm, out_hbm, nb_v, idx_v, out_v, sem_nb, sem_a):
    wid = lax.axis_index("s") * _NC + lax.axis_index("c")
    base = wid * _RPW
    c_nb = pltpu.async_copy(nb_hbm, nb_v.at[pl.ds(0, _NUM_ROWS)], sem_nb)
    c_a = pltpu.async_copy(idx_hbm.at[pl.ds(base, _RPW)], idx_v, sem_a)

    zeros = jnp.zeros((_L,), jnp.int32)
    ones = jnp.ones((_L,), jnp.int32)
    lanes = lax.broadcasted_iota(jnp.int32, (_L,), 0)

    @plsc.parallel_loop(0, _RPW * _CH, unroll=8)
    def _zero(i):
        out_v[i // _CH, pl.ds((i % _CH) * _L, _L)] = zeros

    c_nb.wait()
    c_a.wait()

    @plsc.parallel_loop(0, _RPW * _CH, unroll=16)
    def _scatter(i):
        rl = i // _CH
        g = i % _CH
        nb_r = nb_v[pl.ds(base + rl, _L)][0]
        rows = jnp.full((_L,), rl, jnp.int32)
        mask = (lanes + g * _L) < nb_r
        ids = idx_v[rl, pl.ds(g * _L, _L)]
        plsc.store_scatter(out_v, [rows, ids], ones, mask=mask)
    pltpu.sync_copy(out_v, out_hbm.at[pl.ds(base, _RPW)])


def kernel(kv_num_blocks, kv_indices):
    mesh = plsc.VectorSubcoreMesh(
        core_axis_name="c", subcore_axis_name="s", num_cores=_NC)
    f = pl.kernel(
        _sc_body,
        out_type=jax.ShapeDtypeStruct((_NUM_ROWS, _NUM_COLS), jnp.int32),
        mesh=mesh,
        scratch_types=[
            pltpu.VMEM((_NUM_ROWS + _L,), jnp.int32),
            pltpu.VMEM((_RPW, _NUM_COLS), jnp.int32),
            pltpu.VMEM((_RPW, _NUM_COLS), jnp.int32),
            pltpu.SemaphoreType.DMA,
            pltpu.SemaphoreType.DMA,
        ],
        compiler_params=pltpu.CompilerParams(needs_layout_passes=False),
    )
    return f(kv_num_blocks, kv_indices)


# speedup vs baseline: 1.0189x; 1.0006x over previous
import jax
import jax.numpy as jnp
from jax import lax
from jax.experimental import pallas as pl
from jax.experimental.pallas import tpu as pltpu
from jax.experimental.pallas import tpu_sc as plsc

_NUM_ROWS = 128
_NUM_COLS = 128
_NC = 1
_NS = 16
_NW = _NC * _NS
_RPW = _NUM_ROWS // _NW
_L = 16
_CH = _NUM_COLS // _L


def _sc_body(nb_hbm, idx_hbm, out_hbm, nb_v, idx_v, out_v, sem_nb, sem_a):
    wid = lax.axis_index("s") * _NC + lax.axis_index("c")
    base = wid * _RPW
    c_nb = pltpu.async_copy(nb_hbm, nb_v.at[pl.ds(0, _NUM_ROWS)], sem_nb)
    c_a = pltpu.async_copy(idx_hbm.at[pl.ds(base, _RPW)], idx_v, sem_a)

    zeros = jnp.zeros((_L,), jnp.int32)
    ones = jnp.ones((_L,), jnp.int32)
    lanes = lax.broadcasted_iota(jnp.int32, (_L,), 0)

    @plsc.parallel_loop(0, _RPW * _CH, unroll=8)
    def _zero(i):
        out_v[i // _CH, pl.ds((i % _CH) * _L, _L)] = zeros

    c_nb.wait()
    c_a.wait()

    @plsc.parallel_loop(0, _RPW * _CH, unroll=16)
    def _scatter(i):
        rl = i // _CH
        g = i % _CH
        nb_r = nb_v[pl.ds(base + rl, _L)][0]
        rows = jnp.full((_L,), rl, jnp.int32)
        mask = (lanes + g * _L) < nb_r
        ids = idx_v[rl, pl.ds(g * _L, _L)]
        plsc.store_scatter(out_v, [rows, ids], ones, mask=mask)
    pltpu.sync_copy(out_v, out_hbm.at[pl.ds(base, _RPW)])


def kernel(kv_num_blocks, kv_indices):
    mesh = plsc.VectorSubcoreMesh(
        core_axis_name="c", subcore_axis_name="s", num_cores=_NC)
    f = pl.kernel(
        _sc_body,
        out_type=jax.ShapeDtypeStruct((_NUM_ROWS, _NUM_COLS), jnp.int32),
        mesh=mesh,
        scratch_types=[
            pltpu.VMEM((_NUM_ROWS + _L,), jnp.int32),
            pltpu.VMEM((_RPW, _NUM_COLS), jnp.int32),
            pltpu.VMEM((_RPW, _NUM_COLS), jnp.int32),
            pltpu.SemaphoreType.DMA,
            pltpu.SemaphoreType.DMA,
        ],
        compiler_params=pltpu.CompilerParams(needs_layout_passes=False),
    )
    return f(kv_num_blocks, kv_indices)
